# baseline (speedup 1.0000x reference)
import functools

import jax
import jax.numpy as jnp
from jax import lax
from jax.experimental import pallas as pl
from jax.experimental.pallas import tpu as pltpu
from jax.experimental.pallas import tpu_sc as plsc

NC = 2
NS = 16
NW = NC * NS

GROUP_B = 8


def _make_kernel(batch, n_fields, embed_dim):
    mesh = plsc.VectorSubcoreMesh(core_axis_name="c", subcore_axis_name="s")
    b_w = batch // NW
    n_groups = b_w // GROUP_B

    @functools.partial(
        pl.kernel,
        out_type=jax.ShapeDtypeStruct((batch, n_fields, embed_dim), jnp.float32),
        mesh=mesh,
        scratch_types=[
            pltpu.VMEM((b_w, n_fields), jnp.int32),
            pltpu.VMEM((2, GROUP_B * n_fields, embed_dim), jnp.float32),
            pltpu.SemaphoreType.DMA,
            pltpu.SemaphoreType.DMA,
            pltpu.SemaphoreType.DMA,
            pltpu.SemaphoreType.DMA,
        ],
        compiler_params=pltpu.CompilerParams(use_tc_tiling_on_sc=False),
    )
    def gather_kernel(table_hbm, idx_hbm, out_hbm, idx_v, rows_v,
                      sem_g0, sem_g1, sem_w0, sem_w1):
        wid = lax.axis_index("s") * NC + lax.axis_index("c")
        b_base = wid * b_w
        sem_g = (sem_g0, sem_g1)
        sem_w = (sem_w0, sem_w1)

        pltpu.sync_copy(idx_hbm.at[pl.ds(b_base, b_w)], idx_v)

        def gath(g, parity, j):
            return pltpu.make_async_copy(
                table_hbm.at[idx_v.at[g * GROUP_B + j]],
                rows_v.at[parity, pl.ds(j * n_fields, n_fields)],
                sem_g[parity],
            )

        def writ(g, parity, j):
            return pltpu.make_async_copy(
                rows_v.at[parity, pl.ds(j * n_fields, n_fields)],
                out_hbm.at[b_base + g * GROUP_B + j],
                sem_w[parity],
            )

        def fire(g, parity):
            for j in range(GROUP_B):
                gath(g, parity, j).start()

        def step(g, parity, fire_ahead):
            for j in range(GROUP_B):
                gath(g, parity, j).wait()
            for j in range(GROUP_B):
                writ(g, parity, j).start()
            if fire_ahead:
                for j in range(GROUP_B):
                    writ(g, parity, j).wait()
                fire(g + 2, parity)

        fire(0, 0)
        fire(1, 1)

        n_reg = n_groups - 3
        n_reg -= n_reg % 2

        def body(i, carry):
            g = i * 2
            step(g, 0, True)
            step(g + 1, 1, True)
            return carry

        lax.fori_loop(0, n_reg // 2, body, 0)

        for g in range(n_reg, n_groups):
            step(g, g % 2, g + 2 < n_groups)
        for g in (n_groups - 2, n_groups - 1):
            for j in range(GROUP_B):
                writ(g, g % 2, j).wait()

    return gather_kernel


def kernel(indices, table):
    batch, n_fields = indices.shape
    vocab, embed_dim = table.shape
    assert batch % (NW * GROUP_B) == 0
    t128 = lax.optimization_barrier(table.reshape(vocab * embed_dim // 128, 128))
    tlin = t128.reshape(vocab, embed_dim)
    out = _make_kernel(batch, n_fields, embed_dim)(tlin, indices)
    y128 = lax.optimization_barrier(
        out.reshape(batch * n_fields * embed_dim // 128, 128)
    )
    return y128.reshape(batch, n_fields, embed_dim)


# speedup vs baseline: 1.1071x; 1.1071x over previous
import functools

import jax
import jax.numpy as jnp
from jax import lax
from jax.experimental import pallas as pl
from jax.experimental.pallas import tpu as pltpu
from jax.experimental.pallas import tpu_sc as plsc

NC = 2
NS = 16
NW = NC * NS

GROUP_B = 8


def _make_kernel(batch, n_fields, embed_dim):
    mesh = plsc.VectorSubcoreMesh(core_axis_name="c", subcore_axis_name="s")
    b_w = batch // NW
    n_groups = b_w // GROUP_B

    @functools.partial(
        pl.kernel,
        out_type=jax.ShapeDtypeStruct((batch, n_fields, embed_dim), jnp.float32),
        mesh=mesh,
        scratch_types=[
            pltpu.VMEM((b_w, n_fields), jnp.int32),
            pltpu.VMEM((2, GROUP_B * n_fields, embed_dim), jnp.float32),
            pltpu.SemaphoreType.DMA,
            pltpu.SemaphoreType.DMA,
            pltpu.SemaphoreType.DMA,
            pltpu.SemaphoreType.DMA,
        ],
        compiler_params=pltpu.CompilerParams(use_tc_tiling_on_sc=False),
    )
    def gather_kernel(table_hbm, idx_hbm, out_hbm, idx_v, rows_v,
                      sem_g0, sem_g1, sem_w0, sem_w1):
        wid = lax.axis_index("s") * NC + lax.axis_index("c")
        b_base = wid * b_w
        sem_g = (sem_g0, sem_g1)
        sem_w = (sem_w0, sem_w1)

        pltpu.sync_copy(idx_hbm.at[pl.ds(b_base, b_w)], idx_v)

        def gath(g, parity, j):
            return pltpu.make_async_copy(
                table_hbm.at[idx_v.at[g * GROUP_B + j]],
                rows_v.at[parity, pl.ds(j * n_fields, n_fields)],
                sem_g[parity],
            )

        def writ(g, parity, j):
            return pltpu.make_async_copy(
                rows_v.at[parity, pl.ds(j * n_fields, n_fields)],
                out_hbm.at[b_base + g * GROUP_B + j],
                sem_w[parity],
            )

        def fire(g, parity):
            for j in range(GROUP_B):
                gath(g, parity, j).start()

        def step(g, parity, fire_ahead):
            for j in range(GROUP_B):
                gath(g, parity, j).wait()
            for j in range(GROUP_B):
                writ(g, parity, j).start()
            if fire_ahead:
                for j in range(GROUP_B):
                    writ(g, parity, j).wait()
                fire(g + 2, parity)

        fire(0, 0)
        fire(1, 1)

        n_reg = n_groups - 3
        n_reg -= n_reg % 2

        def body(i, carry):
            g = i * 2
            step(g, 0, True)
            step(g + 1, 1, True)
            return carry

        lax.fori_loop(0, n_reg // 2, body, 0)

        for g in range(n_reg, n_groups):
            step(g, g % 2, g + 2 < n_groups)
        for g in (n_groups - 2, n_groups - 1):
            for j in range(GROUP_B):
                writ(g, g % 2, j).wait()

    return gather_kernel


TC_BLK = 4096


def _make_compact(vocab, embed_dim):
    fold = 128 // embed_dim
    n_blocks = (vocab + TC_BLK - 1) // TC_BLK

    def body(x_ref, y_ref):
        x = x_ref[...]
        x3 = x.T.reshape(TC_BLK // fold, fold, embed_dim)
        y_ref[...] = jnp.concatenate(
            [x3[:, a, :] for a in range(fold)], axis=1
        )

    return pl.pallas_call(
        body,
        grid=(n_blocks,),
        in_specs=[pl.BlockSpec((embed_dim, TC_BLK), lambda i: (0, i))],
        out_specs=pl.BlockSpec((TC_BLK // fold, 128), lambda i: (i, 0)),
        out_shape=jax.ShapeDtypeStruct((vocab * embed_dim // 128, 128),
                                       jnp.float32),
    )


def kernel(indices, table):
    batch, n_fields = indices.shape
    vocab, embed_dim = table.shape
    assert batch % (NW * GROUP_B) == 0
    t128 = _make_compact(vocab, embed_dim)(table.T)
    tlin = t128.reshape(vocab, embed_dim)
    return _make_kernel(batch, n_fields, embed_dim)(tlin, indices)


# speedup vs baseline: 1.5367x; 1.3880x over previous
import functools

import jax
import jax.numpy as jnp
from jax import lax
from jax.experimental import pallas as pl
from jax.experimental.pallas import tpu as pltpu
from jax.experimental.pallas import tpu_sc as plsc

NC = 2
NS = 16
NW = NC * NS

GROUP_B = 8


def _make_kernel(batch, n_fields, embed_dim):
    mesh = plsc.VectorSubcoreMesh(core_axis_name="c", subcore_axis_name="s")
    b_w = batch // NW
    n_groups = b_w // GROUP_B

    @functools.partial(
        pl.kernel,
        out_type=jax.ShapeDtypeStruct((batch, n_fields, embed_dim), jnp.float32),
        mesh=mesh,
        scratch_types=[
            pltpu.VMEM((b_w, n_fields), jnp.int32),
            pltpu.VMEM((2, GROUP_B * n_fields, embed_dim), jnp.float32),
            pltpu.SemaphoreType.DMA,
            pltpu.SemaphoreType.DMA,
            pltpu.SemaphoreType.DMA,
            pltpu.SemaphoreType.DMA,
        ],
        compiler_params=pltpu.CompilerParams(use_tc_tiling_on_sc=False),
    )
    def gather_kernel(table_hbm, idx_hbm, out_hbm, idx_v, rows_v,
                      sem_g0, sem_g1, sem_w0, sem_w1):
        wid = lax.axis_index("s") * NC + lax.axis_index("c")
        b_base = wid * b_w
        sem_g = (sem_g0, sem_g1)
        sem_w = (sem_w0, sem_w1)

        pltpu.sync_copy(idx_hbm.at[pl.ds(b_base, b_w)], idx_v)

        def gath(g, parity, j):
            return pltpu.make_async_copy(
                table_hbm.at[idx_v.at[g * GROUP_B + j]],
                rows_v.at[parity, pl.ds(j * n_fields, n_fields)],
                sem_g[parity],
            )

        def writ(g, parity, j):
            return pltpu.make_async_copy(
                rows_v.at[parity, pl.ds(j * n_fields, n_fields)],
                out_hbm.at[b_base + g * GROUP_B + j],
                sem_w[parity],
            )

        def fire(g, parity):
            for j in range(GROUP_B):
                gath(g, parity, j).start()

        def step(g, parity, fire_ahead):
            for j in range(GROUP_B):
                gath(g, parity, j).wait()
            for j in range(GROUP_B):
                writ(g, parity, j).start()
            if fire_ahead:
                for j in range(GROUP_B):
                    writ(g, parity, j).wait()
                fire(g + 2, parity)

        fire(0, 0)
        fire(1, 1)

        n_reg = n_groups - 3
        n_reg -= n_reg % 2

        def body(i, carry):
            g = i * 2
            step(g, 0, True)
            step(g + 1, 1, True)
            return carry

        lax.fori_loop(0, n_reg // 2, body, 0)

        for g in range(n_reg, n_groups):
            step(g, g % 2, g + 2 < n_groups)
        for g in (n_groups - 2, n_groups - 1):
            for j in range(GROUP_B):
                writ(g, g % 2, j).wait()

    return gather_kernel


TC_BLK = 4096


def _make_stage(vocab, embed_dim):
    n_blocks = (vocab + TC_BLK - 1) // TC_BLK

    def body(x_ref, y_ref):
        x = x_ref[...]
        for q in range(TC_BLK // 512):
            v = jnp.concatenate(
                [x[:, 512 * q + 128 * u:512 * q + 128 * (u + 1)]
                 for u in range(4)], axis=0)
            y_ref[pl.ds(128 * q, 128), :] = v.T

    return pl.pallas_call(
        body,
        grid=(n_blocks,),
        in_specs=[pl.BlockSpec((embed_dim, TC_BLK), lambda i: (0, i))],
        out_specs=pl.BlockSpec((TC_BLK // 4, 128), lambda i: (i, 0)),
        out_shape=jax.ShapeDtypeStruct((n_blocks * TC_BLK // 4, 128),
                                       jnp.float32),
    )


def kernel(indices, table):
    batch, n_fields = indices.shape
    vocab, embed_dim = table.shape
    assert batch % (NW * GROUP_B) == 0
    assert embed_dim == 32
    t128 = _make_stage(vocab, embed_dim)(table.T)
    tlin = t128.reshape(t128.shape[0] * 4, embed_dim)
    idx_r = (
        jnp.bitwise_and(indices, -512)
        | jnp.left_shift(jnp.bitwise_and(indices, 127), 2)
        | jnp.bitwise_and(jnp.right_shift(indices, 7), 3)
    )
    return _make_kernel(batch, n_fields, embed_dim)(tlin, idx_r)


# speedup vs baseline: 1.8640x; 1.2130x over previous
import functools

import jax
import jax.numpy as jnp
from jax import lax
from jax.experimental import pallas as pl
from jax.experimental.pallas import tpu as pltpu
from jax.experimental.pallas import tpu_sc as plsc

NC = 2
NS = 16
NW = NC * NS

GROUP_B = 8


def _make_kernel(batch, n_fields, embed_dim):
    mesh = plsc.VectorSubcoreMesh(core_axis_name="c", subcore_axis_name="s")
    b_w = batch // NW
    n_groups = b_w // GROUP_B

    @functools.partial(
        pl.kernel,
        out_type=jax.ShapeDtypeStruct((batch, n_fields, embed_dim), jnp.float32),
        mesh=mesh,
        scratch_types=[
            pltpu.VMEM((b_w, n_fields), jnp.int32),
            pltpu.VMEM((2, GROUP_B * n_fields, embed_dim), jnp.float32),
            pltpu.SemaphoreType.DMA,
            pltpu.SemaphoreType.DMA,
            pltpu.SemaphoreType.DMA,
            pltpu.SemaphoreType.DMA,
        ],
        compiler_params=pltpu.CompilerParams(use_tc_tiling_on_sc=False),
    )
    def gather_kernel(table_hbm, idx_hbm, out_hbm, idx_v, rows_v,
                      sem_g0, sem_g1, sem_w0, sem_w1):
        wid = lax.axis_index("s") * NC + lax.axis_index("c")
        b_base = wid * b_w
        sem_g = (sem_g0, sem_g1)
        sem_w = (sem_w0, sem_w1)

        pltpu.sync_copy(idx_hbm.at[pl.ds(b_base, b_w)], idx_v)

        def gath(g, parity, j):
            return pltpu.make_async_copy(
                table_hbm.at[idx_v.at[g * GROUP_B + j]],
                rows_v.at[parity, pl.ds(j * n_fields, n_fields)],
                sem_g[parity],
            )

        def writ(g, parity, j):
            return pltpu.make_async_copy(
                rows_v.at[parity, pl.ds(j * n_fields, n_fields)],
                out_hbm.at[b_base + g * GROUP_B + j],
                sem_w[parity],
            )

        def fire(g, parity):
            for j in range(GROUP_B):
                gath(g, parity, j).start()

        def step(g, parity, fire_ahead):
            for j in range(GROUP_B):
                gath(g, parity, j).wait()
            for j in range(GROUP_B):
                writ(g, parity, j).start()
            if fire_ahead:
                for j in range(GROUP_B):
                    writ(g, parity, j).wait()
                fire(g + 2, parity)

        fire(0, 0)
        fire(1, 1)

        n_reg = n_groups - 3
        n_reg -= n_reg % 2

        def body(i, carry):
            g = i * 2
            step(g, 0, True)
            step(g + 1, 1, True)
            return carry

        lax.fori_loop(0, n_reg // 2, body, 0)

        for g in range(n_reg, n_groups):
            step(g, g % 2, g + 2 < n_groups)
        for g in (n_groups - 2, n_groups - 1):
            for j in range(GROUP_B):
                writ(g, g % 2, j).wait()

    return gather_kernel


TC_BLK = 16384


def _make_stage(vocab, embed_dim):
    n_blocks = (vocab + TC_BLK - 1) // TC_BLK

    def body(x_ref, y_ref):
        x = x_ref[...]
        for q in range(TC_BLK // 512):
            v = jnp.concatenate(
                [x[:, 512 * q + 128 * u:512 * q + 128 * (u + 1)]
                 for u in range(4)], axis=0)
            y_ref[pl.ds(128 * q, 128), :] = v.T

    return pl.pallas_call(
        body,
        grid=(n_blocks,),
        in_specs=[pl.BlockSpec((embed_dim, TC_BLK), lambda i: (0, i))],
        out_specs=pl.BlockSpec((TC_BLK // 4, 128), lambda i: (i, 0)),
        out_shape=jax.ShapeDtypeStruct((n_blocks * TC_BLK // 4, 128),
                                       jnp.float32),
    )


def kernel(indices, table):
    batch, n_fields = indices.shape
    vocab, embed_dim = table.shape
    assert batch % (NW * GROUP_B) == 0
    assert embed_dim == 32
    t128 = _make_stage(vocab, embed_dim)(table.T)
    tlin = t128.reshape(t128.shape[0] * 4, embed_dim)
    idx_r = (
        jnp.bitwise_and(indices, -512)
        | jnp.left_shift(jnp.bitwise_and(indices, 127), 2)
        | jnp.bitwise_and(jnp.right_shift(indices, 7), 3)
    )
    return _make_kernel(batch, n_fields, embed_dim)(tlin, idx_r)


# speedup vs baseline: 1.9288x; 1.0348x over previous
import functools

import jax
import jax.numpy as jnp
from jax import lax
from jax.experimental import pallas as pl
from jax.experimental.pallas import tpu as pltpu
from jax.experimental.pallas import tpu_sc as plsc

NC = 2
NS = 16
NW = NC * NS

GROUP_B = 8


def _make_kernel(batch, n_fields, embed_dim):
    mesh = plsc.VectorSubcoreMesh(core_axis_name="c", subcore_axis_name="s")
    b_w = batch // NW
    n_groups = b_w // GROUP_B

    @functools.partial(
        pl.kernel,
        out_type=jax.ShapeDtypeStruct((batch, n_fields, embed_dim), jnp.float32),
        mesh=mesh,
        scratch_types=[
            pltpu.VMEM((b_w, n_fields), jnp.int32),
            pltpu.VMEM((2, GROUP_B * n_fields, embed_dim), jnp.float32),
            pltpu.SemaphoreType.DMA,
            pltpu.SemaphoreType.DMA,
            pltpu.SemaphoreType.DMA,
            pltpu.SemaphoreType.DMA,
        ],
        compiler_params=pltpu.CompilerParams(use_tc_tiling_on_sc=False),
    )
    def gather_kernel(table_hbm, idx_hbm, out_hbm, idx_v, rows_v,
                      sem_g0, sem_g1, sem_w0, sem_w1):
        wid = lax.axis_index("s") * NC + lax.axis_index("c")
        b_base = wid * b_w
        sem_g = (sem_g0, sem_g1)
        sem_w = (sem_w0, sem_w1)

        pltpu.sync_copy(idx_hbm.at[pl.ds(b_base, b_w)], idx_v)

        def gath(g, parity, j):
            return pltpu.make_async_copy(
                table_hbm.at[idx_v.at[g * GROUP_B + j]],
                rows_v.at[parity, pl.ds(j * n_fields, n_fields)],
                sem_g[parity],
            )

        def writ(g, parity, j):
            return pltpu.make_async_copy(
                rows_v.at[parity, pl.ds(j * n_fields, n_fields)],
                out_hbm.at[b_base + g * GROUP_B + j],
                sem_w[parity],
            )

        def fire(g, parity):
            for j in range(GROUP_B):
                gath(g, parity, j).start()

        def step(g, parity, fire_ahead):
            for j in range(GROUP_B):
                gath(g, parity, j).wait()
            for j in range(GROUP_B):
                writ(g, parity, j).start()
            if fire_ahead:
                for j in range(GROUP_B):
                    writ(g, parity, j).wait()
                fire(g + 2, parity)

        fire(0, 0)
        fire(1, 1)

        n_reg = n_groups - 3
        n_reg -= n_reg % 2

        def body(i, carry):
            g = i * 2
            step(g, 0, True)
            step(g + 1, 1, True)
            return carry

        lax.fori_loop(0, n_reg // 2, body, 0)

        for g in range(n_reg, n_groups):
            step(g, g % 2, g + 2 < n_groups)
        for g in (n_groups - 2, n_groups - 1):
            for j in range(GROUP_B):
                writ(g, g % 2, j).wait()

    return gather_kernel


TC_BLK = 32768


def _make_stage(vocab, embed_dim):
    n_blocks = (vocab + TC_BLK - 1) // TC_BLK

    def body(x_ref, y_ref):
        x = x_ref[...]
        for q in range(TC_BLK // 512):
            v = jnp.concatenate(
                [x[:, 512 * q + 128 * u:512 * q + 128 * (u + 1)]
                 for u in range(4)], axis=0)
            y_ref[pl.ds(128 * q, 128), :] = v.T

    return pl.pallas_call(
        body,
        grid=(n_blocks,),
        in_specs=[pl.BlockSpec((embed_dim, TC_BLK), lambda i: (0, i))],
        out_specs=pl.BlockSpec((TC_BLK // 4, 128), lambda i: (i, 0)),
        out_shape=jax.ShapeDtypeStruct((n_blocks * TC_BLK // 4, 128),
                                       jnp.float32),
    )


def kernel(indices, table):
    batch, n_fields = indices.shape
    vocab, embed_dim = table.shape
    assert batch % (NW * GROUP_B) == 0
    assert embed_dim == 32
    t128 = _make_stage(vocab, embed_dim)(table.T)
    tlin = t128.reshape(t128.shape[0] * 4, embed_dim)
    idx_r = (
        jnp.bitwise_and(indices, -512)
        | jnp.left_shift(jnp.bitwise_and(indices, 127), 2)
        | jnp.bitwise_and(jnp.right_shift(indices, 7), 3)
    )
    return _make_kernel(batch, n_fields, embed_dim)(tlin, idx_r)


# speedup vs baseline: 1.9385x; 1.0050x over previous
import functools

import jax
import jax.numpy as jnp
from jax import lax
from jax.experimental import pallas as pl
from jax.experimental.pallas import tpu as pltpu
from jax.experimental.pallas import tpu_sc as plsc

NC = 2
NS = 16
NW = NC * NS

GROUP_B = 8


def _make_kernel(batch, n_fields, embed_dim):
    mesh = plsc.VectorSubcoreMesh(core_axis_name="c", subcore_axis_name="s")
    b_w = batch // NW
    n_groups = b_w // GROUP_B

    @functools.partial(
        pl.kernel,
        out_type=jax.ShapeDtypeStruct((batch, n_fields, embed_dim), jnp.float32),
        mesh=mesh,
        scratch_types=[
            pltpu.VMEM((b_w, n_fields), jnp.int32),
            pltpu.VMEM((2, GROUP_B * n_fields, embed_dim), jnp.float32),
            pltpu.SemaphoreType.DMA,
            pltpu.SemaphoreType.DMA,
            pltpu.SemaphoreType.DMA,
            pltpu.SemaphoreType.DMA,
        ],
        compiler_params=pltpu.CompilerParams(use_tc_tiling_on_sc=False),
    )
    def gather_kernel(table_hbm, idx_hbm, out_hbm, idx_v, rows_v,
                      sem_g0, sem_g1, sem_w0, sem_w1):
        wid = lax.axis_index("s") * NC + lax.axis_index("c")
        b_base = wid * b_w
        sem_g = (sem_g0, sem_g1)
        sem_w = (sem_w0, sem_w1)

        pltpu.sync_copy(idx_hbm.at[pl.ds(b_base, b_w)], idx_v)

        def gath(g, parity, j):
            return pltpu.make_async_copy(
                table_hbm.at[idx_v.at[g * GROUP_B + j]],
                rows_v.at[parity, pl.ds(j * n_fields, n_fields)],
                sem_g[parity],
            )

        def writ(g, parity, j):
            return pltpu.make_async_copy(
                rows_v.at[parity, pl.ds(j * n_fields, n_fields)],
                out_hbm.at[b_base + g * GROUP_B + j],
                sem_w[parity],
            )

        def fire(g, parity):
            for j in range(GROUP_B):
                gath(g, parity, j).start()

        def step(g, parity, fire_ahead):
            for j in range(GROUP_B):
                gath(g, parity, j).wait()
            for j in range(GROUP_B):
                writ(g, parity, j).start()
            if fire_ahead:
                for j in range(GROUP_B):
                    writ(g, parity, j).wait()
                fire(g + 2, parity)

        fire(0, 0)
        fire(1, 1)

        n_reg = n_groups - 3
        n_reg -= n_reg % 2

        def body(i, carry):
            g = i * 2
            step(g, 0, True)
            step(g + 1, 1, True)
            return carry

        lax.fori_loop(0, n_reg // 2, body, 0)

        for g in range(n_reg, n_groups):
            step(g, g % 2, g + 2 < n_groups)
        for g in (n_groups - 2, n_groups - 1):
            for j in range(GROUP_B):
                writ(g, g % 2, j).wait()

    return gather_kernel


TC_BLK = 65536


def _make_stage(vocab, embed_dim):
    n_blocks = (vocab + TC_BLK - 1) // TC_BLK

    def body(x_ref, y_ref):
        x = x_ref[...]
        for q in range(TC_BLK // 512):
            v = jnp.concatenate(
                [x[:, 512 * q + 128 * u:512 * q + 128 * (u + 1)]
                 for u in range(4)], axis=0)
            y_ref[pl.ds(128 * q, 128), :] = v.T

    return pl.pallas_call(
        body,
        grid=(n_blocks,),
        in_specs=[pl.BlockSpec((embed_dim, TC_BLK), lambda i: (0, i))],
        out_specs=pl.BlockSpec((TC_BLK // 4, 128), lambda i: (i, 0)),
        out_shape=jax.ShapeDtypeStruct((n_blocks * TC_BLK // 4, 128),
                                       jnp.float32),
    )


def kernel(indices, table):
    batch, n_fields = indices.shape
    vocab, embed_dim = table.shape
    assert batch % (NW * GROUP_B) == 0
    assert embed_dim == 32
    t128 = _make_stage(vocab, embed_dim)(table.T)
    tlin = t128.reshape(t128.shape[0] * 4, embed_dim)
    idx_r = (
        jnp.bitwise_and(indices, -512)
        | jnp.left_shift(jnp.bitwise_and(indices, 127), 2)
        | jnp.bitwise_and(jnp.right_shift(indices, 7), 3)
    )
    return _make_kernel(batch, n_fields, embed_dim)(tlin, idx_r)


# speedup vs baseline: 1.9842x; 1.0236x over previous
import functools

import jax
import jax.numpy as jnp
from jax import lax
from jax.experimental import pallas as pl
from jax.experimental.pallas import tpu as pltpu
from jax.experimental.pallas import tpu_sc as plsc

NC = 2
NS = 16
NW = NC * NS

GROUP_B = 16


def _make_kernel(batch, n_fields, embed_dim):
    mesh = plsc.VectorSubcoreMesh(core_axis_name="c", subcore_axis_name="s")
    b_w = batch // NW
    n_groups = b_w // GROUP_B

    @functools.partial(
        pl.kernel,
        out_type=jax.ShapeDtypeStruct((batch, n_fields, embed_dim), jnp.float32),
        mesh=mesh,
        scratch_types=[
            pltpu.VMEM((b_w, n_fields), jnp.int32),
            pltpu.VMEM((2, GROUP_B * n_fields, embed_dim), jnp.float32),
            pltpu.SemaphoreType.DMA,
            pltpu.SemaphoreType.DMA,
            pltpu.SemaphoreType.DMA,
            pltpu.SemaphoreType.DMA,
        ],
        compiler_params=pltpu.CompilerParams(use_tc_tiling_on_sc=False),
    )
    def gather_kernel(table_hbm, idx_hbm, out_hbm, idx_v, rows_v,
                      sem_g0, sem_g1, sem_w0, sem_w1):
        wid = lax.axis_index("s") * NC + lax.axis_index("c")
        b_base = wid * b_w
        sem_g = (sem_g0, sem_g1)
        sem_w = (sem_w0, sem_w1)

        pltpu.sync_copy(idx_hbm.at[pl.ds(b_base, b_w)], idx_v)

        def gath(g, parity, j):
            return pltpu.make_async_copy(
                table_hbm.at[idx_v.at[g * GROUP_B + j]],
                rows_v.at[parity, pl.ds(j * n_fields, n_fields)],
                sem_g[parity],
            )

        def writ(g, parity, j):
            return pltpu.make_async_copy(
                rows_v.at[parity, pl.ds(j * n_fields, n_fields)],
                out_hbm.at[b_base + g * GROUP_B + j],
                sem_w[parity],
            )

        def fire(g, parity):
            for j in range(GROUP_B):
                gath(g, parity, j).start()

        def step(g, parity, fire_ahead):
            for j in range(GROUP_B):
                gath(g, parity, j).wait()
            for j in range(GROUP_B):
                writ(g, parity, j).start()
            if fire_ahead:
                for j in range(GROUP_B):
                    writ(g, parity, j).wait()
                fire(g + 2, parity)

        fire(0, 0)
        fire(1, 1)

        n_reg = n_groups - 3
        n_reg -= n_reg % 2

        def body(i, carry):
            g = i * 2
            step(g, 0, True)
            step(g + 1, 1, True)
            return carry

        lax.fori_loop(0, n_reg // 2, body, 0)

        for g in range(n_reg, n_groups):
            step(g, g % 2, g + 2 < n_groups)
        for g in (n_groups - 2, n_groups - 1):
            for j in range(GROUP_B):
                writ(g, g % 2, j).wait()

    return gather_kernel


TC_BLK = 65536


def _make_stage(vocab, embed_dim):
    n_blocks = (vocab + TC_BLK - 1) // TC_BLK

    def body(x_ref, y_ref):
        x = x_ref[...]
        for q in range(TC_BLK // 512):
            v = jnp.concatenate(
                [x[:, 512 * q + 128 * u:512 * q + 128 * (u + 1)]
                 for u in range(4)], axis=0)
            y_ref[pl.ds(128 * q, 128), :] = v.T

    return pl.pallas_call(
        body,
        grid=(n_blocks,),
        in_specs=[pl.BlockSpec((embed_dim, TC_BLK), lambda i: (0, i))],
        out_specs=pl.BlockSpec((TC_BLK // 4, 128), lambda i: (i, 0)),
        out_shape=jax.ShapeDtypeStruct((n_blocks * TC_BLK // 4, 128),
                                       jnp.float32),
    )


def kernel(indices, table):
    batch, n_fields = indices.shape
    vocab, embed_dim = table.shape
    assert batch % (NW * GROUP_B) == 0
    assert embed_dim == 32
    t128 = _make_stage(vocab, embed_dim)(table.T)
    tlin = t128.reshape(t128.shape[0] * 4, embed_dim)
    idx_r = (
        jnp.bitwise_and(indices, -512)
        | jnp.left_shift(jnp.bitwise_and(indices, 127), 2)
        | jnp.bitwise_and(jnp.right_shift(indices, 7), 3)
    )
    return _make_kernel(batch, n_fields, embed_dim)(tlin, idx_r)


# speedup vs baseline: 1.9865x; 1.0012x over previous
import functools

import jax
import jax.numpy as jnp
from jax import lax
from jax.experimental import pallas as pl
from jax.experimental.pallas import tpu as pltpu
from jax.experimental.pallas import tpu_sc as plsc

NC = 2
NS = 16
NW = NC * NS

GROUP_B = 32


def _make_kernel(batch, n_fields, embed_dim):
    mesh = plsc.VectorSubcoreMesh(core_axis_name="c", subcore_axis_name="s")
    b_w = batch // NW
    n_groups = b_w // GROUP_B

    @functools.partial(
        pl.kernel,
        out_type=jax.ShapeDtypeStruct((batch, n_fields, embed_dim), jnp.float32),
        mesh=mesh,
        scratch_types=[
            pltpu.VMEM((b_w, n_fields), jnp.int32),
            pltpu.VMEM((2, GROUP_B * n_fields, embed_dim), jnp.float32),
            pltpu.SemaphoreType.DMA,
            pltpu.SemaphoreType.DMA,
            pltpu.SemaphoreType.DMA,
            pltpu.SemaphoreType.DMA,
        ],
        compiler_params=pltpu.CompilerParams(use_tc_tiling_on_sc=False),
    )
    def gather_kernel(table_hbm, idx_hbm, out_hbm, idx_v, rows_v,
                      sem_g0, sem_g1, sem_w0, sem_w1):
        wid = lax.axis_index("s") * NC + lax.axis_index("c")
        b_base = wid * b_w
        sem_g = (sem_g0, sem_g1)
        sem_w = (sem_w0, sem_w1)

        pltpu.sync_copy(idx_hbm.at[pl.ds(b_base, b_w)], idx_v)

        def gath(g, parity, j):
            return pltpu.make_async_copy(
                table_hbm.at[idx_v.at[g * GROUP_B + j]],
                rows_v.at[parity, pl.ds(j * n_fields, n_fields)],
                sem_g[parity],
            )

        def writ(g, parity, j):
            return pltpu.make_async_copy(
                rows_v.at[parity, pl.ds(j * n_fields, n_fields)],
                out_hbm.at[b_base + g * GROUP_B + j],
                sem_w[parity],
            )

        def fire(g, parity):
            for j in range(GROUP_B):
                gath(g, parity, j).start()

        def step(g, parity, fire_ahead):
            for j in range(GROUP_B):
                gath(g, parity, j).wait()
            for j in range(GROUP_B):
                writ(g, parity, j).start()
            if fire_ahead:
                for j in range(GROUP_B):
                    writ(g, parity, j).wait()
                fire(g + 2, parity)

        fire(0, 0)
        fire(1, 1)

        n_reg = n_groups - 3
        n_reg -= n_reg % 2

        def body(i, carry):
            g = i * 2
            step(g, 0, True)
            step(g + 1, 1, True)
            return carry

        lax.fori_loop(0, n_reg // 2, body, 0)

        for g in range(n_reg, n_groups):
            step(g, g % 2, g + 2 < n_groups)
        for g in (n_groups - 2, n_groups - 1):
            for j in range(GROUP_B):
                writ(g, g % 2, j).wait()

    return gather_kernel


TC_BLK = 65536


def _make_stage(vocab, embed_dim):
    n_blocks = (vocab + TC_BLK - 1) // TC_BLK

    def body(x_ref, y_ref):
        x = x_ref[...]
        for q in range(TC_BLK // 512):
            v = jnp.concatenate(
                [x[:, 512 * q + 128 * u:512 * q + 128 * (u + 1)]
                 for u in range(4)], axis=0)
            y_ref[pl.ds(128 * q, 128), :] = v.T

    return pl.pallas_call(
        body,
        grid=(n_blocks,),
        in_specs=[pl.BlockSpec((embed_dim, TC_BLK), lambda i: (0, i))],
        out_specs=pl.BlockSpec((TC_BLK // 4, 128), lambda i: (i, 0)),
        out_shape=jax.ShapeDtypeStruct((n_blocks * TC_BLK // 4, 128),
                                       jnp.float32),
    )


def kernel(indices, table):
    batch, n_fields = indices.shape
    vocab, embed_dim = table.shape
    assert batch % (NW * GROUP_B) == 0
    assert embed_dim == 32
    t128 = _make_stage(vocab, embed_dim)(table.T)
    tlin = t128.reshape(t128.shape[0] * 4, embed_dim)
    idx_r = (
        jnp.bitwise_and(indices, -512)
        | jnp.left_shift(jnp.bitwise_and(indices, 127), 2)
        | jnp.bitwise_and(jnp.right_shift(indices, 7), 3)
    )
    return _make_kernel(batch, n_fields, embed_dim)(tlin, idx_r)
